# baseline (speedup 1.0000x reference)
import functools

import jax
import jax.numpy as jnp
from jax import lax
from jax.experimental import pallas as pl
from jax.experimental.pallas import tpu as pltpu
from jax.experimental.pallas import tpu_sc as plsc

_N = 100000
_D = 128
_H = 64
_NUM_MOL = 512

_NW = 16
_CHUNK = 128
_ROWS_PER_W = 49
_SLAB = _ROWS_PER_W * _CHUNK
_NCHUNK = 1
_CATOMS = _NW * _SLAB
_NPAD = _NCHUNK * _CATOMS
_BLK = 8 * _SLAB
_STEPS = _CATOMS // _BLK
_ACC = 520


def _mlp_body(x_ref, w1_ref, b1_ref, w2t_ref, ab_ref, out_ref):
    h = jnp.dot(x_ref[...], w1_ref[...], preferred_element_type=jnp.float32)
    h = h + b1_ref[...]
    h = h * jax.nn.sigmoid(h)
    v = lax.dot_general(w2t_ref[...], h, (((1,), (1,)), ((), ())),
                        preferred_element_type=jnp.float32)
    out_ref[...] = (v + ab_ref[0, 0]).reshape(1, 1, _BLK)


def _make_mlp(chunk):
    base = chunk * _STEPS
    return pl.pallas_call(
        _mlp_body,
        grid=(_STEPS,),
        in_specs=[
            pl.BlockSpec((_BLK, _D), lambda i: (i + base, 0)),
            pl.BlockSpec((_D, _H), lambda i: (0, 0)),
            pl.BlockSpec((1, _H), lambda i: (0, 0)),
            pl.BlockSpec((1, _H), lambda i: (0, 0)),
            pl.BlockSpec(memory_space=pltpu.SMEM),
        ],
        out_specs=pl.BlockSpec((1, 1, _BLK), lambda i: (i, 0, 0)),
        out_shape=jax.ShapeDtypeStruct((_STEPS, 1, _BLK), jnp.float32),
    )


_mlp_calls = [_make_mlp(c) for c in range(_NCHUNK)]


def _seg_body(vals_hbm, idx_hbm, init_hbm, out_hbm, vals_v, idx_v, acc_sh,
              sem_v, sem_i, sem_s):
    s = lax.axis_index("s")

    vcopy = pltpu.async_copy(vals_hbm.at[s], vals_v, sem_v)
    icopy = pltpu.async_copy(idx_hbm.at[s], idx_v, sem_i)

    @pl.when(s == 0)
    def _init():
        pltpu.sync_copy(init_hbm, acc_sh)

    plsc.subcore_barrier()
    vcopy.wait()
    icopy.wait()

    def _fire(j, carry):
        pltpu.async_copy(vals_v.at[j], acc_sh.at[idx_v.at[j]], sem_s,
                         add=True)
        return carry

    lax.fori_loop(0, _ROWS_PER_W, _fire, 0)
    pltpu.make_async_copy(vals_hbm.at[s], vals_v, sem_s).wait()

    plsc.subcore_barrier()

    @pl.when(s == 0)
    def _emit():
        pltpu.sync_copy(acc_sh, out_hbm)


_seg_call = functools.partial(
    pl.kernel,
    out_type=jax.ShapeDtypeStruct((_ACC,), jnp.float32),
    mesh=plsc.VectorSubcoreMesh(core_axis_name="c", subcore_axis_name="s",
                                num_cores=1),
    scratch_types=[
        pltpu.VMEM((_ROWS_PER_W, _CHUNK), jnp.float32),
        pltpu.VMEM((_ROWS_PER_W, _CHUNK), jnp.int32),
        pltpu.VMEM_SHARED((_ACC,), jnp.float32),
        pltpu.SemaphoreType.DMA,
        pltpu.SemaphoreType.DMA,
        pltpu.SemaphoreType.DMA,
    ],
)(_seg_body)


def kernel(x_scalar, x_spherical, coord, batch_index, W1, b1, W2, b2,
           node_bias, graph_bias):
    ab = (b2[0] + node_bias).reshape(1, 1).astype(jnp.float32)
    b1r = b1.reshape(1, _H)
    w2t = W2.reshape(1, _H)

    idx_pad = jnp.concatenate(
        [batch_index, jnp.full((_NPAD - _N,), _NUM_MOL, dtype=jnp.int32)]
    ).reshape(_NCHUNK, _NW, _ROWS_PER_W, _CHUNK)

    acc = jnp.full((_ACC,), graph_bias, dtype=jnp.float32)
    for c in range(_NCHUNK):
        atom = _mlp_calls[c](x_scalar, W1, b1r, w2t, ab)
        vals = atom.reshape(_NW, _ROWS_PER_W, _CHUNK)
        acc = _seg_call(vals, idx_pad[c], acc)

    res = acc[:_NUM_MOL].reshape(_NUM_MOL, 1)
    neg_grad = jnp.zeros_like(coord)
    return res, neg_grad


# speedup vs baseline: 1.0458x; 1.0458x over previous
import functools

import jax
import jax.numpy as jnp
from jax import lax
from jax.experimental import pallas as pl
from jax.experimental.pallas import tpu as pltpu
from jax.experimental.pallas import tpu_sc as plsc

_N = 100000
_D = 128
_H = 64
_NUM_MOL = 512

_NW = 16
_CHUNK = 128
_ROWS_PER_W = 49
_SLAB = _ROWS_PER_W * _CHUNK
_NCHUNK = 1
_CATOMS = _NW * _SLAB
_NPAD = _NCHUNK * _CATOMS
_BLK = 4 * _SLAB
_STEPS = _CATOMS // _BLK
_ACC = 520


def _mlp_body(x_ref, w1_ref, b1_ref, w2t_ref, ab_ref, out_ref):
    h = jnp.dot(x_ref[...], w1_ref[...], preferred_element_type=jnp.float32)
    h = h + b1_ref[...]
    h = h * jax.nn.sigmoid(h)
    v = lax.dot_general(w2t_ref[...], h, (((1,), (1,)), ((), ())),
                        preferred_element_type=jnp.float32)
    out_ref[...] = (v + ab_ref[0, 0]).reshape(1, 1, _BLK)


def _make_mlp(chunk):
    base = chunk * _STEPS
    return pl.pallas_call(
        _mlp_body,
        grid=(_STEPS,),
        in_specs=[
            pl.BlockSpec((_BLK, _D), lambda i: (i + base, 0)),
            pl.BlockSpec((_D, _H), lambda i: (0, 0)),
            pl.BlockSpec((1, _H), lambda i: (0, 0)),
            pl.BlockSpec((1, _H), lambda i: (0, 0)),
            pl.BlockSpec(memory_space=pltpu.SMEM),
        ],
        out_specs=pl.BlockSpec((1, 1, _BLK), lambda i: (i, 0, 0)),
        out_shape=jax.ShapeDtypeStruct((_STEPS, 1, _BLK), jnp.float32),
    )


_mlp_calls = [_make_mlp(c) for c in range(_NCHUNK)]


def _seg_body(vals_hbm, idx_hbm, init_hbm, out_hbm, vals_v, idx_v, acc_sh,
              sem_v, sem_i, sem_s):
    s = lax.axis_index("s")

    vcopy = pltpu.async_copy(vals_hbm.at[s], vals_v, sem_v)
    icopy = pltpu.async_copy(idx_hbm.at[s], idx_v, sem_i)

    @pl.when(s == 0)
    def _init():
        pltpu.sync_copy(init_hbm, acc_sh)

    plsc.subcore_barrier()
    vcopy.wait()
    icopy.wait()

    pltpu.sync_copy(vals_v, acc_sh.at[idx_v], add=True)

    plsc.subcore_barrier()

    @pl.when(s == 0)
    def _emit():
        pltpu.sync_copy(acc_sh, out_hbm)


_seg_call = functools.partial(
    pl.kernel,
    out_type=jax.ShapeDtypeStruct((_ACC,), jnp.float32),
    mesh=plsc.VectorSubcoreMesh(core_axis_name="c", subcore_axis_name="s",
                                num_cores=1),
    scratch_types=[
        pltpu.VMEM((_SLAB,), jnp.float32),
        pltpu.VMEM((_SLAB,), jnp.int32),
        pltpu.VMEM_SHARED((_ACC,), jnp.float32),
        pltpu.SemaphoreType.DMA,
        pltpu.SemaphoreType.DMA,
        pltpu.SemaphoreType.DMA,
    ],
)(_seg_body)


def kernel(x_scalar, x_spherical, coord, batch_index, W1, b1, W2, b2,
           node_bias, graph_bias):
    ab = (b2[0] + node_bias).reshape(1, 1).astype(jnp.float32)
    b1r = b1.reshape(1, _H)
    w2t = W2.reshape(1, _H)

    idx_pad = jnp.concatenate(
        [batch_index, jnp.full((_NPAD - _N,), _NUM_MOL, dtype=jnp.int32)]
    ).reshape(_NCHUNK, _NW, _SLAB)

    acc = jnp.full((_ACC,), graph_bias, dtype=jnp.float32)
    for c in range(_NCHUNK):
        atom = _mlp_calls[c](x_scalar, W1, b1r, w2t, ab)
        vals = atom.reshape(_NW, _SLAB)
        acc = _seg_call(vals, idx_pad[c], acc)

    res = acc[:_NUM_MOL].reshape(_NUM_MOL, 1)
    neg_grad = jnp.zeros_like(coord)
    return res, neg_grad
